# baseline (speedup 1.0000x reference)
import functools

import jax
import jax.numpy as jnp
from jax import lax
from jax.experimental import pallas as pl
from jax.experimental.pallas import tpu as pltpu
from jax.experimental.pallas import tpu_sc as plsc

_DIM = 64
_L = 16384
_NG = 1000000
_NC = 2
_NS = 16
_NW = _NC * _NS
_CHUNK = _L // _NW
_LANES = 16
_ROWW = 128
_SPLIT = 3904
_NROWB = 3912
_WBLK = 488
_C = _WBLK * _ROWW
_HSTEP = 8
_ZSPAN = 240


def _weights_body(g_ref, c_ref, w_ref):
    g = g_ref[...]
    c = c_ref[...]
    t = jnp.log1p(jnp.where(g >= 0, c, 0.0))
    w_ref[...] = t * (1.0 / jnp.sum(t))


_weights = pl.pallas_call(
    _weights_body,
    out_shape=jax.ShapeDtypeStruct((128, 128), jnp.float32),
)

_mesh = plsc.VectorSubcoreMesh(
    core_axis_name="c", subcore_axis_name="s", num_cores=_NC, num_subcores=_NS
)


def _make_scatter(nrows, lo):
    alloc = nrows + 8
    last_span = alloc - 15 * _ZSPAN
    out_span = nrows - 15 * _ZSPAN

    @functools.partial(
        pl.kernel,
        out_type=jax.ShapeDtypeStruct((_NC, nrows, _ROWW), jnp.float32),
        mesh=_mesh,
        compiler_params=pltpu.CompilerParams(needs_layout_passes=False),
        scratch_types=[
            pltpu.VMEM((_CHUNK,), jnp.int32),
            pltpu.VMEM((_CHUNK,), jnp.float32),
            pltpu.VMEM((_CHUNK,), jnp.int32),
            pltpu.VMEM((_CHUNK, _ROWW), jnp.float32),
            pltpu.VMEM_SHARED((alloc, _ROWW), jnp.float32),
        ],
    )
    def _scatter(gids_hbm, w_hbm, dense_hbm, idx_v, w_v, ci_v, contrib_v, shared):
        core = lax.axis_index("c")
        sub = lax.axis_index("s")
        wid = sub * _NC + core

        pltpu.sync_copy(gids_hbm.at[wid], idx_v)
        pltpu.sync_copy(w_hbm.at[wid], w_v)

        lane = lax.iota(jnp.int32, _LANES)
        zvec = jnp.zeros((_LANES,), jnp.float32)

        def zfill(r, _):
            for c in range(_ROWW // _LANES):
                contrib_v[r, pl.ds(c * _LANES, _LANES)] = zvec
            return 0

        lax.fori_loop(0, _CHUNK, zfill, 0)

        @pl.when(sub < 15)
        def _():
            pltpu.sync_copy(
                contrib_v.at[pl.ds(0, _ZSPAN)],
                shared.at[pl.ds(sub * _ZSPAN, _ZSPAN)],
            )

        @pl.when(sub == 15)
        def _():
            pltpu.sync_copy(
                contrib_v.at[pl.ds(0, last_span)],
                shared.at[pl.ds(15 * _ZSPAN, last_span)],
            )

        def build(b, _):
            kbase = b * _LANES
            gvec = idx_v[pl.ds(kbase, _LANES)]
            local = lax.shift_right_logical(gvec, 7) - lo
            ok = (local >= 0) & (local < nrows)
            ci_v[pl.ds(kbase, _LANES)] = jnp.where(ok, local, nrows)
            wvec = w_v[pl.ds(kbase, _LANES)]
            plsc.store_scatter(contrib_v, [kbase + lane, gvec & (_ROWW - 1)], wvec)
            return 0

        lax.fori_loop(0, _CHUNK // _LANES, build, 0)

        plsc.subcore_barrier()
        pltpu.sync_copy(contrib_v, shared.at[ci_v], add=True)
        plsc.subcore_barrier()

        @pl.when(sub < 15)
        def _():
            pltpu.sync_copy(
                shared.at[pl.ds(sub * _ZSPAN, _ZSPAN)],
                dense_hbm.at[core, pl.ds(sub * _ZSPAN, _ZSPAN)],
            )

        @pl.when(sub == 15)
        def _():
            pltpu.sync_copy(
                shared.at[pl.ds(15 * _ZSPAN, out_span)],
                dense_hbm.at[core, pl.ds(15 * _ZSPAN, out_span)],
            )

    return _scatter


_scatter_a = _make_scatter(_SPLIT, 0)
_scatter_b = _make_scatter(_NROWB, _SPLIT)


def _mv_body(w_ref, t_ref, o_ref):
    @pl.when(pl.program_id(0) == 0)
    def _():
        o_ref[...] = jnp.zeros_like(o_ref)

    o_ref[...] += jax.lax.dot_general(
        t_ref[...], w_ref[...].reshape(_C),
        dimension_numbers=(((1,), (0,)), ((), ())),
        preferred_element_type=jnp.float32,
    )


def _make_matvec(block0):
    return pl.pallas_call(
        _mv_body,
        grid=(_HSTEP,),
        in_specs=[
            pl.BlockSpec((_WBLK, _ROWW), lambda i: (i, 0)),
            pl.BlockSpec((_DIM, _C), lambda i, b=block0: (0, b + i)),
        ],
        out_specs=pl.BlockSpec((_DIM,), lambda i: (0,)),
        out_shape=jax.ShapeDtypeStruct((_DIM,), jnp.float32),
    )


_matvec_a = _make_matvec(0)
_matvec_b = _make_matvec(_HSTEP)


def _mv_tail_body(w_ref, t_ref, o_ref):
    o_ref[...] = jax.lax.dot_general(
        t_ref[...], w_ref[...],
        dimension_numbers=(((1,), (0,)), ((), ())),
        preferred_element_type=jnp.float32,
    )


_mv_tail = pl.pallas_call(
    _mv_tail_body,
    out_shape=jax.ShapeDtypeStruct((_DIM,), jnp.float32),
)

_TAILBASE = 2 * _HSTEP * _C
_TAIL = _NG - _TAILBASE


@jax.jit
def kernel(gids, cnts, table):
    gids = gids.astype(jnp.int32)
    g2 = gids.reshape(_NW, _CHUNK)
    w2d = _weights(gids.reshape(128, 128), cnts.reshape(128, 128))
    w2 = w2d.reshape(_NW, _CHUNK)
    dense_a = _scatter_a(g2, w2)
    dense_b = _scatter_b(g2, w2)
    wsum_a = dense_a[0] + dense_a[1]
    wsum_b = dense_b[0] + dense_b[1]
    tt = table.T
    out_a = _matvec_a(wsum_a, tt)
    out_b = _matvec_b(wsum_b, tt)
    out_tail = _mv_tail(
        lax.slice(wsum_b.reshape(_NROWB * _ROWW), (_HSTEP * _C,),
                  (_HSTEP * _C + _TAIL,)),
        lax.slice(tt, (0, _TAILBASE), (_DIM, _NG)),
    )
    return out_a + out_b + out_tail


# speedup vs baseline: 1.0568x; 1.0568x over previous
import functools

import jax
import jax.numpy as jnp
from jax import lax
from jax.experimental import pallas as pl
from jax.experimental.pallas import tpu as pltpu
from jax.experimental.pallas import tpu_sc as plsc

_DIM = 64
_L = 16384
_NG = 1000000
_NC = 2
_NS = 16
_NW = _NC * _NS
_CHUNK = _L // _NW
_LANES = 16
_NROW = _NG // _LANES
_ZROW = 3904
_ZREM = _NROW - _NS * _ZROW
_WBLK = 488
_C = _WBLK * 128
_NSTEP = 16
_ALIGNED = _NSTEP * _C
_ROWW = 128
_NROW2 = 7816
_NGPAD = _NROW2 * _ROWW
_ZROW2 = 488
_ZREM2 = _NROW2 - _NS * _ZROW2


def _weights_body(g_ref, c_ref, w_ref):
    g = g_ref[...]
    c = c_ref[...]
    t = jnp.log1p(jnp.where(g >= 0, c, 0.0))
    w_ref[...] = t * (1.0 / jnp.sum(t))


_weights = pl.pallas_call(
    _weights_body,
    out_shape=jax.ShapeDtypeStruct((128, 128), jnp.float32),
)

_mesh = plsc.VectorSubcoreMesh(
    core_axis_name="c", subcore_axis_name="s", num_cores=_NC, num_subcores=_NS
)


@functools.partial(
    pl.kernel,
    out_type=(
        jax.ShapeDtypeStruct((_NROW2, _ROWW), jnp.float32),
        jax.ShapeDtypeStruct((_NROW2, _ROWW), jnp.float32),
    ),
    mesh=_mesh,
    compiler_params=pltpu.CompilerParams(needs_layout_passes=False),
    scratch_types=[
        pltpu.VMEM((_CHUNK,), jnp.int32),
        pltpu.VMEM((_CHUNK,), jnp.float32),
        pltpu.VMEM((_CHUNK,), jnp.int32),
        pltpu.VMEM((_CHUNK, _ROWW), jnp.float32),
        pltpu.VMEM_SHARED((_NROW2, _ROWW), jnp.float32),
    ],
)
def _sc_scatter(gids_hbm, w_hbm, dense0_hbm, dense1_hbm,
                idx_v, w_v, ci_v, contrib_v, shared):
    core = lax.axis_index("c")
    sub = lax.axis_index("s")
    wid = sub * _NC + core

    pltpu.sync_copy(gids_hbm.at[wid], idx_v)
    pltpu.sync_copy(w_hbm.at[wid], w_v)

    lane = lax.iota(jnp.int32, _LANES)
    zvec = jnp.zeros((_LANES,), jnp.float32)

    def zfill(r, _):
        for c in range(_ROWW // _LANES):
            contrib_v[r, pl.ds(c * _LANES, _LANES)] = zvec
        return 0

    lax.fori_loop(0, _CHUNK, zfill, 0)

    pltpu.sync_copy(
        contrib_v.at[pl.ds(0, _ZROW2)],
        shared.at[pl.ds(sub * _ZROW2, _ZROW2)],
    )

    @pl.when(sub == 0)
    def _():
        pltpu.sync_copy(
            contrib_v.at[pl.ds(0, _ZREM2)],
            shared.at[pl.ds(_NS * _ZROW2, _ZREM2)],
        )

    def build(b, _):
        kbase = b * _LANES
        gvec = idx_v[pl.ds(kbase, _LANES)]
        ci_v[pl.ds(kbase, _LANES)] = lax.shift_right_logical(gvec, 7)
        wvec = w_v[pl.ds(kbase, _LANES)]
        rows = kbase + lane
        cols = gvec & (_ROWW - 1)
        plsc.store_scatter(contrib_v, [rows, cols], wvec)
        return 0

    lax.fori_loop(0, _CHUNK // _LANES, build, 0)

    plsc.subcore_barrier()
    pltpu.sync_copy(contrib_v, shared.at[ci_v], add=True)
    plsc.subcore_barrier()

    @pl.when(core == 0)
    def _():
        pltpu.sync_copy(
            shared.at[pl.ds(sub * _ZROW2, _ZROW2)],
            dense0_hbm.at[pl.ds(sub * _ZROW2, _ZROW2)],
        )

        @pl.when(sub == 0)
        def _():
            pltpu.sync_copy(
                shared.at[pl.ds(_NS * _ZROW2, _ZREM2)],
                dense0_hbm.at[pl.ds(_NS * _ZROW2, _ZREM2)],
            )

    @pl.when(core == 1)
    def _():
        pltpu.sync_copy(
            shared.at[pl.ds(sub * _ZROW2, _ZROW2)],
            dense1_hbm.at[pl.ds(sub * _ZROW2, _ZROW2)],
        )

        @pl.when(sub == 0)
        def _():
            pltpu.sync_copy(
                shared.at[pl.ds(_NS * _ZROW2, _ZREM2)],
                dense1_hbm.at[pl.ds(_NS * _ZROW2, _ZREM2)],
            )


def _mv_body(w0_ref, w1_ref, t_ref, o_ref):
    @pl.when(pl.program_id(0) == 0)
    def _():
        o_ref[...] = jnp.zeros_like(o_ref)

    wsum = (w0_ref[...] + w1_ref[...]).reshape(_C)
    o_ref[...] += jax.lax.dot_general(
        t_ref[...], wsum,
        dimension_numbers=(((1,), (0,)), ((), ())),
        preferred_element_type=jnp.float32,
    )


_matvec = pl.pallas_call(
    _mv_body,
    grid=(_NSTEP,),
    in_specs=[
        pl.BlockSpec((_WBLK, _ROWW), lambda i: (i, 0)),
        pl.BlockSpec((_WBLK, _ROWW), lambda i: (i, 0)),
        pl.BlockSpec((_DIM, _C), lambda i: (0, i)),
    ],
    out_specs=pl.BlockSpec((_DIM,), lambda i: (0,)),
    out_shape=jax.ShapeDtypeStruct((_DIM,), jnp.float32),
)


def _mv_tail_body(w0_ref, w1_ref, t_ref, o_ref):
    o_ref[...] = jax.lax.dot_general(
        t_ref[...], w0_ref[...] + w1_ref[...],
        dimension_numbers=(((1,), (0,)), ((), ())),
        preferred_element_type=jnp.float32,
    )


_mv_tail = pl.pallas_call(
    _mv_tail_body,
    out_shape=jax.ShapeDtypeStruct((_DIM,), jnp.float32),
)


@jax.jit
def kernel(gids, cnts, table):
    gids = gids.astype(jnp.int32)
    w2d = _weights(gids.reshape(128, 128), cnts.reshape(128, 128))
    dense0, dense1 = _sc_scatter(
        gids.reshape(_NW, _CHUNK),
        w2d.reshape(_NW, _CHUNK),
    )
    tt = table.T
    out_main = _matvec(dense0, dense1, tt)
    out_tail = _mv_tail(
        lax.slice(dense0.reshape(_NGPAD), (_ALIGNED,), (_NG,)),
        lax.slice(dense1.reshape(_NGPAD), (_ALIGNED,), (_NG,)),
        lax.slice(tt, (0, _ALIGNED), (_DIM, _NG)),
    )
    return out_main + out_tail
